# Initial kernel scaffold; baseline (speedup 1.0000x reference)
import jax
import jax.numpy as jnp
from jax.experimental import pallas as pl


def kernel(x, idx_token, agg_weight, mask):
    raise NotImplementedError("write your pallas kernel here")


# speedup vs baseline: 9.3325x; 9.3325x over previous
import math

import jax
import jax.numpy as jnp
from jax.experimental import pallas as pl
from jax.experimental.pallas import tpu as pltpu

_TR = 256
_INTERPRET = False
_HI = jax.lax.Precision.HIGHEST


def _dens_kernel(xT_ref, x_ref, xt_ref, noise_ref, d_ref, den_ref, rmax_ref):
    c, n = xT_ref.shape[1], xT_ref.shape[2]
    tr = xt_ref.shape[1]
    xT = xT_ref[0]
    xt = xt_ref[0]
    xf = x_ref[0]
    sq_all = jnp.sum(xf * xf, axis=1, keepdims=True)
    sq_row = jnp.transpose(sq_all)
    sq_t = jnp.sum(xt * xt, axis=1, keepdims=True)
    g = jax.lax.dot_general(xt, xT, (((1,), (0,)), ((), ())),
                            preferred_element_type=jnp.float32)
    d2 = sq_t + sq_row - 2.0 * g
    dt = jnp.sqrt(jnp.maximum(d2, 0.0)) / (c ** 0.5)
    d_ref[0] = dt
    rmax_ref[0] = jnp.max(dt, axis=1, keepdims=True)
    iota_j = jax.lax.broadcasted_iota(jnp.int32, (tr, n), 1)
    work = dt
    acc = jnp.zeros((tr, 1), jnp.float32)
    for _ in range(5):
        cur = jnp.min(work, axis=1, keepdims=True)
        acc = acc + cur * cur
        sel = jnp.where(work == cur, iota_j, n)
        first = jnp.min(sel, axis=1, keepdims=True)
        work = jnp.where(iota_j == first, jnp.inf, work)
    den_ref[0] = jnp.exp(-(acc / 5.0)) + noise_ref[0]


def _score_kernel(d_ref, den_ref, dent_ref, rmax_ref, score_ref):
    dt = d_ref[0]
    den = den_ref[0]
    den_t = dent_ref[0]
    den_row = jnp.transpose(den)
    gmax = jnp.max(rmax_ref[0])
    tmp = jnp.where(den_row > den_t, dt, gmax)
    dist_t = jnp.min(tmp, axis=1, keepdims=True)
    score_ref[0] = dist_t * den_t


def _rank_kernel(score_ref, st_ref, rank_ref):
    score = score_ref[0]
    n = score.shape[0]
    ti = pl.program_id(1)
    tr = st_ref.shape[1]
    s_t = st_ref[0]
    s_row = jnp.transpose(score)
    row_id = jax.lax.broadcasted_iota(jnp.int32, (tr, 1), 0) + ti * tr
    col_id = jax.lax.broadcasted_iota(jnp.int32, (tr, n), 1)
    beats = (s_row > s_t) | ((s_row == s_t) & (col_id < row_id))
    rank_ref[0] = jnp.sum(beats.astype(jnp.int32), axis=1, keepdims=True)


def _assign_kernel(d_ref, rank_ref, rankt_ref, cn_ref, idc_ref):
    dt = d_ref[0]
    rank = rank_ref[0]
    cn = cn_ref[0]
    n = dt.shape[1]
    rank_t = rankt_ref[0]
    rank_row_f = jnp.transpose(rank.astype(jnp.float32))
    is_c = rank_row_f < cn
    dmask = jnp.where(is_c, dt, jnp.inf)
    dmin = jnp.min(dmask, axis=1, keepdims=True)
    cand = jnp.where(is_c & (dt == dmin), rank_row_f, float(n))
    amin = jnp.min(cand, axis=1, keepdims=True).astype(jnp.int32)
    idc_ref[0] = jnp.where(rank_t < cn, rank_t, amin)


def _merge_kernel(x_ref, idc_ref, cn_ref, xm_ref, nw_ref):
    x = x_ref[0]
    idc = idc_ref[0]
    cn = cn_ref[0]
    idc_row_f = jnp.transpose(idc.astype(jnp.float32))
    iota_c = jax.lax.broadcasted_iota(
        jnp.int32, (cn, idc.shape[0]), 0).astype(jnp.float32)
    ohT = (idc_row_f == iota_c).astype(jnp.float32)
    counts = jnp.sum(ohT, axis=1, keepdims=True)
    inv = 1.0 / (counts + 1e-6)
    nw_row = jnp.sum(ohT * inv, axis=0, keepdims=True)
    xm_ref[0] = jax.lax.dot_general(ohT * nw_row, x, (((1,), (0,)), ((), ())),
                                    precision=_HI,
                                    preferred_element_type=jnp.float32)
    nw_ref[0] = nw_row


def _gather_kernel(it_ref, agg_ref, idc_ref, nw_ref, itn_ref, aggn_ref):
    it_t = it_ref[0]
    idc = idc_ref[0]
    nw_row = nw_ref[0]
    n = idc.shape[0]
    tr = it_t.shape[0]
    idc_row_f = jnp.transpose(idc.astype(jnp.float32))
    col_f = jax.lax.broadcasted_iota(jnp.int32, (tr, n), 1).astype(jnp.float32)
    eq = it_t.astype(jnp.float32) == col_f
    itn = jnp.sum(jnp.where(eq, idc_row_f, 0.0), axis=1, keepdims=True)
    wt = jnp.sum(jnp.where(eq, nw_row, 0.0), axis=1, keepdims=True)
    itn_ref[0] = itn.astype(jnp.int32)
    aggn_ref[0] = agg_ref[0] * wt


def kernel(x, idx_token, agg_weight, mask):
    b, n, c = x.shape
    cn = max(math.ceil(n * 0.25), 1)
    t = n // _TR
    f32 = jnp.float32

    noise = (jax.random.uniform(jax.random.key(42), (b, n), dtype=f32)
             * 1e-6).reshape(b, n, 1)
    xT = jnp.transpose(x, (0, 2, 1))
    it3 = idx_token.astype(jnp.int32).reshape(b, n, 1)

    D, den, rmax = pl.pallas_call(
        _dens_kernel,
        grid=(b, t),
        in_specs=[
            pl.BlockSpec((1, c, n), lambda bi, ti: (bi, 0, 0)),
            pl.BlockSpec((1, n, c), lambda bi, ti: (bi, 0, 0)),
            pl.BlockSpec((1, _TR, c), lambda bi, ti: (bi, ti, 0)),
            pl.BlockSpec((1, _TR, 1), lambda bi, ti: (bi, ti, 0)),
        ],
        out_specs=[
            pl.BlockSpec((1, _TR, n), lambda bi, ti: (bi, ti, 0)),
            pl.BlockSpec((1, _TR, 1), lambda bi, ti: (bi, ti, 0)),
            pl.BlockSpec((1, _TR, 1), lambda bi, ti: (bi, ti, 0)),
        ],
        out_shape=[
            jax.ShapeDtypeStruct((b, n, n), f32),
            jax.ShapeDtypeStruct((b, n, 1), f32),
            jax.ShapeDtypeStruct((b, n, 1), f32),
        ],
        interpret=_INTERPRET,
    )(xT, x, x, noise)

    score = pl.pallas_call(
        _score_kernel,
        grid=(b, t),
        in_specs=[
            pl.BlockSpec((1, _TR, n), lambda bi, ti: (bi, ti, 0)),
            pl.BlockSpec((1, n, 1), lambda bi, ti: (bi, 0, 0)),
            pl.BlockSpec((1, _TR, 1), lambda bi, ti: (bi, ti, 0)),
            pl.BlockSpec((1, n, 1), lambda bi, ti: (bi, 0, 0)),
        ],
        out_specs=pl.BlockSpec((1, _TR, 1), lambda bi, ti: (bi, ti, 0)),
        out_shape=jax.ShapeDtypeStruct((b, n, 1), f32),
        interpret=_INTERPRET,
    )(D, den, den, rmax)

    rank = pl.pallas_call(
        _rank_kernel,
        grid=(b, t),
        in_specs=[
            pl.BlockSpec((1, n, 1), lambda bi, ti: (bi, 0, 0)),
            pl.BlockSpec((1, _TR, 1), lambda bi, ti: (bi, ti, 0)),
        ],
        out_specs=pl.BlockSpec((1, _TR, 1), lambda bi, ti: (bi, ti, 0)),
        out_shape=jax.ShapeDtypeStruct((b, n, 1), jnp.int32),
        interpret=_INTERPRET,
    )(score, score)

    def _assign(d_ref, rank_ref, rankt_ref, idc_ref):
        _assign_kernel(d_ref, rank_ref, rankt_ref, [cn], idc_ref)

    idc = pl.pallas_call(
        _assign,
        grid=(b, t),
        in_specs=[
            pl.BlockSpec((1, _TR, n), lambda bi, ti: (bi, ti, 0)),
            pl.BlockSpec((1, n, 1), lambda bi, ti: (bi, 0, 0)),
            pl.BlockSpec((1, _TR, 1), lambda bi, ti: (bi, ti, 0)),
        ],
        out_specs=pl.BlockSpec((1, _TR, 1), lambda bi, ti: (bi, ti, 0)),
        out_shape=jax.ShapeDtypeStruct((b, n, 1), jnp.int32),
        interpret=_INTERPRET,
    )(D, rank, rank)

    def _merge(x_ref, idc_ref, xm_ref, nw_ref):
        _merge_kernel(x_ref, idc_ref, [cn], xm_ref, nw_ref)

    x_merged, nw = pl.pallas_call(
        _merge,
        grid=(b,),
        in_specs=[
            pl.BlockSpec((1, n, c), lambda bi: (bi, 0, 0)),
            pl.BlockSpec((1, n, 1), lambda bi: (bi, 0, 0)),
        ],
        out_specs=[
            pl.BlockSpec((1, cn, c), lambda bi: (bi, 0, 0)),
            pl.BlockSpec((1, 1, n), lambda bi: (bi, 0, 0)),
        ],
        out_shape=[
            jax.ShapeDtypeStruct((b, cn, c), f32),
            jax.ShapeDtypeStruct((b, 1, n), f32),
        ],
        interpret=_INTERPRET,
    )(x, idc)

    itn, aggn = pl.pallas_call(
        _gather_kernel,
        grid=(b, t),
        in_specs=[
            pl.BlockSpec((1, _TR, 1), lambda bi, ti: (bi, ti, 0)),
            pl.BlockSpec((1, _TR, 1), lambda bi, ti: (bi, ti, 0)),
            pl.BlockSpec((1, n, 1), lambda bi, ti: (bi, 0, 0)),
            pl.BlockSpec((1, 1, n), lambda bi, ti: (bi, 0, 0)),
        ],
        out_specs=[
            pl.BlockSpec((1, _TR, 1), lambda bi, ti: (bi, ti, 0)),
            pl.BlockSpec((1, _TR, 1), lambda bi, ti: (bi, ti, 0)),
        ],
        out_shape=[
            jax.ShapeDtypeStruct((b, n, 1), jnp.int32),
            jax.ShapeDtypeStruct((b, n, 1), f32),
        ],
        interpret=_INTERPRET,
    )(it3, agg_weight, idc, nw)

    return (x_merged, itn.reshape(b, n), aggn, idc.reshape(b, n))


# speedup vs baseline: 9.3347x; 1.0002x over previous
import math

import jax
import jax.numpy as jnp
from jax.experimental import pallas as pl
from jax.experimental.pallas import tpu as pltpu

_TR = 256
_INTERPRET = False
_HI = jax.lax.Precision.HIGHEST


def _dens_kernel(xT_ref, sq_ref, sqt_ref, xt_ref, d_ref, nn_ref, rmax_ref):
    c, n = xT_ref.shape[1], xT_ref.shape[2]
    tr = xt_ref.shape[1]
    xT = xT_ref[0]
    xt = xt_ref[0]
    sq_row = sq_ref[0]
    sq_t = sqt_ref[0]
    g = jax.lax.dot_general(xt, xT, (((1,), (0,)), ((), ())),
                            preferred_element_type=jnp.float32)
    d2 = sq_t + sq_row - 2.0 * g
    dt = jnp.sqrt(jnp.maximum(d2, 0.0)) / (c ** 0.5)
    d_ref[0] = dt
    rmax_ref[0] = jnp.max(dt, axis=1, keepdims=True)
    iota_j = jax.lax.broadcasted_iota(jnp.int32, (tr, n), 1)
    work = dt
    vals = []
    for _ in range(5):
        cur = jnp.min(work, axis=1, keepdims=True)
        vals.append(cur)
        sel = jnp.where(work == cur, iota_j, n)
        first = jnp.min(sel, axis=1, keepdims=True)
        work = jnp.where(iota_j == first, jnp.inf, work)
    vals.append(jnp.zeros((tr, 3), jnp.float32))
    nn_ref[0] = jnp.concatenate(vals, axis=1)


def _score_kernel(d_ref, den_ref, dent_ref, rmax_ref, score_ref):
    dt = d_ref[0]
    den = den_ref[0]
    den_t = dent_ref[0]
    den_row = jnp.transpose(den)
    gmax = jnp.max(rmax_ref[0])
    tmp = jnp.where(den_row > den_t, dt, gmax)
    dist_t = jnp.min(tmp, axis=1, keepdims=True)
    score_ref[0] = dist_t * den_t


def _rank_kernel(score_ref, st_ref, rank_ref):
    score = score_ref[0]
    n = score.shape[0]
    ti = pl.program_id(1)
    tr = st_ref.shape[1]
    s_t = st_ref[0]
    s_row = jnp.transpose(score)
    row_id = jax.lax.broadcasted_iota(jnp.int32, (tr, 1), 0) + ti * tr
    col_id = jax.lax.broadcasted_iota(jnp.int32, (tr, n), 1)
    beats = (s_row > s_t) | ((s_row == s_t) & (col_id < row_id))
    rank_ref[0] = jnp.sum(beats.astype(jnp.int32), axis=1, keepdims=True)


def _assign_kernel(d_ref, rank_ref, rankt_ref, cn_ref, idc_ref):
    dt = d_ref[0]
    rank = rank_ref[0]
    cn = cn_ref[0]
    n = dt.shape[1]
    rank_t = rankt_ref[0]
    rank_row_f = jnp.transpose(rank.astype(jnp.float32))
    is_c = rank_row_f < cn
    dmask = jnp.where(is_c, dt, jnp.inf)
    dmin = jnp.min(dmask, axis=1, keepdims=True)
    cand = jnp.where(is_c & (dt == dmin), rank_row_f, float(n))
    amin = jnp.min(cand, axis=1, keepdims=True).astype(jnp.int32)
    idc_ref[0] = jnp.where(rank_t < cn, rank_t, amin)


def _merge_kernel(x_ref, idc_ref, cn_ref, xm_ref, nw_ref):
    x = x_ref[0]
    idc = idc_ref[0]
    cn = cn_ref[0]
    idc_row_f = jnp.transpose(idc.astype(jnp.float32))
    iota_c = jax.lax.broadcasted_iota(
        jnp.int32, (cn, idc.shape[0]), 0).astype(jnp.float32)
    ohT = (idc_row_f == iota_c).astype(jnp.float32)
    counts = jnp.sum(ohT, axis=1, keepdims=True)
    inv = 1.0 / (counts + 1e-6)
    nw_row = jnp.sum(ohT * inv, axis=0, keepdims=True)
    xm_ref[0] = jax.lax.dot_general(ohT * nw_row, x, (((1,), (0,)), ((), ())),
                                    precision=_HI,
                                    preferred_element_type=jnp.float32)
    nw_ref[0] = nw_row


def _gather_kernel(it_ref, agg_ref, idc_ref, nw_ref, itn_ref, aggn_ref):
    it_t = it_ref[0]
    idc = idc_ref[0]
    nw_row = nw_ref[0]
    n = idc.shape[0]
    tr = it_t.shape[0]
    idc_row_f = jnp.transpose(idc.astype(jnp.float32))
    col_f = jax.lax.broadcasted_iota(jnp.int32, (tr, n), 1).astype(jnp.float32)
    eq = it_t.astype(jnp.float32) == col_f
    itn = jnp.sum(jnp.where(eq, idc_row_f, 0.0), axis=1, keepdims=True)
    wt = jnp.sum(jnp.where(eq, nw_row, 0.0), axis=1, keepdims=True)
    itn_ref[0] = itn.astype(jnp.int32)
    aggn_ref[0] = agg_ref[0] * wt


def kernel(x, idx_token, agg_weight, mask):
    b, n, c = x.shape
    cn = max(math.ceil(n * 0.25), 1)
    t = n // _TR
    f32 = jnp.float32

    noise = jax.random.uniform(jax.random.key(42), (b, n), dtype=f32) * 1e-6
    xT = jnp.transpose(x, (0, 2, 1))
    it3 = idx_token.astype(jnp.int32).reshape(b, n, 1)
    sq = jnp.sum(x * x, axis=-1)
    sq_row = sq.reshape(b, 1, n)
    sq_col = sq.reshape(b, n, 1)

    D, nn, rmax = pl.pallas_call(
        _dens_kernel,
        grid=(b, t),
        in_specs=[
            pl.BlockSpec((1, c, n), lambda bi, ti: (bi, 0, 0)),
            pl.BlockSpec((1, 1, n), lambda bi, ti: (bi, 0, 0)),
            pl.BlockSpec((1, _TR, 1), lambda bi, ti: (bi, ti, 0)),
            pl.BlockSpec((1, _TR, c), lambda bi, ti: (bi, ti, 0)),
        ],
        out_specs=[
            pl.BlockSpec((1, _TR, n), lambda bi, ti: (bi, ti, 0)),
            pl.BlockSpec((1, _TR, 8), lambda bi, ti: (bi, ti, 0)),
            pl.BlockSpec((1, _TR, 1), lambda bi, ti: (bi, ti, 0)),
        ],
        out_shape=[
            jax.ShapeDtypeStruct((b, n, n), f32),
            jax.ShapeDtypeStruct((b, n, 8), f32),
            jax.ShapeDtypeStruct((b, n, 1), f32),
        ],
        interpret=_INTERPRET,
    )(xT, sq_row, sq_col, x)

    den = (jnp.exp(-(nn[..., :5] ** 2).mean(axis=-1)) + noise).reshape(b, n, 1)

    score = pl.pallas_call(
        _score_kernel,
        grid=(b, t),
        in_specs=[
            pl.BlockSpec((1, _TR, n), lambda bi, ti: (bi, ti, 0)),
            pl.BlockSpec((1, n, 1), lambda bi, ti: (bi, 0, 0)),
            pl.BlockSpec((1, _TR, 1), lambda bi, ti: (bi, ti, 0)),
            pl.BlockSpec((1, n, 1), lambda bi, ti: (bi, 0, 0)),
        ],
        out_specs=pl.BlockSpec((1, _TR, 1), lambda bi, ti: (bi, ti, 0)),
        out_shape=jax.ShapeDtypeStruct((b, n, 1), f32),
        interpret=_INTERPRET,
    )(D, den, den, rmax)

    rank = pl.pallas_call(
        _rank_kernel,
        grid=(b, t),
        in_specs=[
            pl.BlockSpec((1, n, 1), lambda bi, ti: (bi, 0, 0)),
            pl.BlockSpec((1, _TR, 1), lambda bi, ti: (bi, ti, 0)),
        ],
        out_specs=pl.BlockSpec((1, _TR, 1), lambda bi, ti: (bi, ti, 0)),
        out_shape=jax.ShapeDtypeStruct((b, n, 1), jnp.int32),
        interpret=_INTERPRET,
    )(score, score)

    def _assign(d_ref, rank_ref, rankt_ref, idc_ref):
        _assign_kernel(d_ref, rank_ref, rankt_ref, [cn], idc_ref)

    idc = pl.pallas_call(
        _assign,
        grid=(b, t),
        in_specs=[
            pl.BlockSpec((1, _TR, n), lambda bi, ti: (bi, ti, 0)),
            pl.BlockSpec((1, n, 1), lambda bi, ti: (bi, 0, 0)),
            pl.BlockSpec((1, _TR, 1), lambda bi, ti: (bi, ti, 0)),
        ],
        out_specs=pl.BlockSpec((1, _TR, 1), lambda bi, ti: (bi, ti, 0)),
        out_shape=jax.ShapeDtypeStruct((b, n, 1), jnp.int32),
        interpret=_INTERPRET,
    )(D, rank, rank)

    def _merge(x_ref, idc_ref, xm_ref, nw_ref):
        _merge_kernel(x_ref, idc_ref, [cn], xm_ref, nw_ref)

    x_merged, nw = pl.pallas_call(
        _merge,
        grid=(b,),
        in_specs=[
            pl.BlockSpec((1, n, c), lambda bi: (bi, 0, 0)),
            pl.BlockSpec((1, n, 1), lambda bi: (bi, 0, 0)),
        ],
        out_specs=[
            pl.BlockSpec((1, cn, c), lambda bi: (bi, 0, 0)),
            pl.BlockSpec((1, 1, n), lambda bi: (bi, 0, 0)),
        ],
        out_shape=[
            jax.ShapeDtypeStruct((b, cn, c), f32),
            jax.ShapeDtypeStruct((b, 1, n), f32),
        ],
        interpret=_INTERPRET,
    )(x, idc)

    itn, aggn = pl.pallas_call(
        _gather_kernel,
        grid=(b, t),
        in_specs=[
            pl.BlockSpec((1, _TR, 1), lambda bi, ti: (bi, ti, 0)),
            pl.BlockSpec((1, _TR, 1), lambda bi, ti: (bi, ti, 0)),
            pl.BlockSpec((1, n, 1), lambda bi, ti: (bi, 0, 0)),
            pl.BlockSpec((1, 1, n), lambda bi, ti: (bi, 0, 0)),
        ],
        out_specs=[
            pl.BlockSpec((1, _TR, 1), lambda bi, ti: (bi, ti, 0)),
            pl.BlockSpec((1, _TR, 1), lambda bi, ti: (bi, ti, 0)),
        ],
        out_shape=[
            jax.ShapeDtypeStruct((b, n, 1), jnp.int32),
            jax.ShapeDtypeStruct((b, n, 1), f32),
        ],
        interpret=_INTERPRET,
    )(it3, agg_weight, idc, nw)

    return (x_merged, itn.reshape(b, n), aggn, idc.reshape(b, n))
